# Initial kernel scaffold; baseline (speedup 1.0000x reference)
import math

import jax
import jax.numpy as jnp
from jax import lax
from jax.experimental import pallas as pl
from jax.experimental.pallas import tpu as pltpu
from jax.experimental.pallas import tpu_sc as plsc

_N = 10000
_E = 320000
_D = 128
_INV_SQRT_D = 1.0 / math.sqrt(128.0)

_NC = 2
_NS = 16
_NW = _NC * _NS
_EPW = _E // _NW
_C = 80
_NCHUNK = _EPW // _C
_AW = 144

_ROWBLK = 1000


def _proj_body(x_ref, wfc, bfc, wq, bq, wk, bk, wv, bv, ws, bs,
               q_ref, k_ref, v_ref, skip_ref):
    x = x_ref[...]
    h = jnp.maximum(
        jnp.dot(x, wfc[...], preferred_element_type=jnp.float32) + bfc[...],
        0.0)
    q_ref[...] = jnp.dot(h, wq[...], preferred_element_type=jnp.float32) + bq[...]
    k_ref[...] = jnp.dot(h, wk[...], preferred_element_type=jnp.float32) + bk[...]
    v_ref[...] = jnp.dot(h, wv[...], preferred_element_type=jnp.float32) + bv[...]
    skip_ref[...] = jnp.dot(h, ws[...], preferred_element_type=jnp.float32) + bs[...]


def _projections(x, wfc_t, bfc, wq_t, bq, wk_t, bk, wv_t, bv, ws_t, bs):
    w_spec = pl.BlockSpec((_D, _D), lambda i: (0, 0))
    b_spec = pl.BlockSpec((1, _D), lambda i: (0, 0))
    r_spec = pl.BlockSpec((_ROWBLK, _D), lambda i: (i, 0))
    out_sd = jax.ShapeDtypeStruct((_N, _D), jnp.float32)
    return pl.pallas_call(
        _proj_body,
        grid=(_N // _ROWBLK,),
        in_specs=[r_spec] + [w_spec, b_spec] * 5,
        out_specs=[r_spec] * 4,
        out_shape=[out_sd] * 4,
    )(x, wfc_t, bfc, wq_t, bq, wk_t, bk, wv_t, bv, ws_t, bs)


def _edge_body(q_hbm, k_hbm, v_hbm, src_hbm, dst_hbm, out_hbm,
               qr, kr, vr, sr, srcc, dstc, lg, acc):
    cid = lax.axis_index("c")
    sid = lax.axis_index("s")
    wid = cid * _NS + sid

    @pl.loop(0, _C)
    def _(e):
        for c in range(_AW // 16):
            sr[e, pl.ds(c * 16, 16)] = jnp.zeros((16,), jnp.float32)

    @pl.loop(0, 39)
    def _(i):
        pltpu.sync_copy(sr.at[pl.ds(0, 16)],
                        acc.at[pl.ds(sid * 624 + i * 16, 16)])

    @pl.when(sid == 0)
    def _():
        pltpu.sync_copy(sr.at[pl.ds(0, 16)], acc.at[pl.ds(9984, 16)])

    plsc.subcore_barrier()

    base = wid * _EPW

    @pl.loop(0, _NCHUNK)
    def _(ci):
        off = base + ci * _C
        pltpu.sync_copy(src_hbm.at[pl.ds(off, _C)], srcc)
        pltpu.sync_copy(dst_hbm.at[pl.ds(off, _C)], dstc)
        pltpu.sync_copy(q_hbm.at[dstc], qr)
        pltpu.sync_copy(k_hbm.at[srcc], kr)
        pltpu.sync_copy(v_hbm.at[srcc], vr)

        @pl.loop(0, _C)
        def _(e):
            p = qr[e, pl.ds(0, 16)] * kr[e, pl.ds(0, 16)]
            for c in range(1, 8):
                p = p + qr[e, pl.ds(c * 16, 16)] * kr[e, pl.ds(c * 16, 16)]
            lg[e] = jnp.sum(p)

        @pl.loop(0, _C // 16)
        def _(i):
            lv = lg[pl.ds(i * 16, 16)]
            lg[pl.ds(i * 16, 16)] = jnp.exp(lv * _INV_SQRT_D)

        @pl.loop(0, _C)
        def _(e):
            ex = lg[e]
            for c in range(8):
                sr[e, pl.ds(c * 16, 16)] = vr[e, pl.ds(c * 16, 16)] * ex
            sr[e, 128] = ex

        pltpu.sync_copy(sr, acc.at[dstc], add=True)

    plsc.subcore_barrier()

    @pl.loop(0, 39)
    def _(i):
        r0 = sid * 624 + i * 16
        pltpu.sync_copy(acc.at[pl.ds(r0, 16)],
                        out_hbm.at[cid].at[pl.ds(r0, 16)])

    @pl.when(sid == 0)
    def _():
        pltpu.sync_copy(acc.at[pl.ds(9984, 16)],
                        out_hbm.at[cid].at[pl.ds(9984, 16)])


def _edge_phase(q, k, v, src, dst):
    mesh = plsc.VectorSubcoreMesh(core_axis_name="c", subcore_axis_name="s")
    ker = pl.kernel(
        _edge_body,
        out_type=jax.ShapeDtypeStruct((_NC, _N, _AW), jnp.float32),
        mesh=mesh,
        scratch_types=[
            pltpu.VMEM((_C, _D), jnp.float32),
            pltpu.VMEM((_C, _D), jnp.float32),
            pltpu.VMEM((_C, _D), jnp.float32),
            pltpu.VMEM((_C, _AW), jnp.float32),
            pltpu.VMEM((_C,), jnp.int32),
            pltpu.VMEM((_C,), jnp.int32),
            pltpu.VMEM((_C,), jnp.float32),
            pltpu.VMEM_SHARED((_N, _AW), jnp.float32),
        ],
    )
    return ker(q, k, v, src, dst)


def _final_body(nd0_ref, nd1_ref, skip_ref, o_ref):
    nd = nd0_ref[...] + nd1_ref[...]
    num = nd[:, :_D]
    den = nd[:, _D:_D + 1]
    o_ref[...] = num / (den + 1e-16) + skip_ref[...]


def _finalize(nd0, nd1, skip):
    a_spec = pl.BlockSpec((_ROWBLK, _AW), lambda i: (i, 0))
    r_spec = pl.BlockSpec((_ROWBLK, _D), lambda i: (i, 0))
    return pl.pallas_call(
        _final_body,
        grid=(_N // _ROWBLK,),
        in_specs=[a_spec, a_spec, r_spec],
        out_specs=r_spec,
        out_shape=jax.ShapeDtypeStruct((_N, _D), jnp.float32),
    )(nd0, nd1, skip)


@jax.jit
def kernel(x, edge_index, W_fc, b_fc, Wq, bq, Wk, bk, Wv, bv, Wskip, bskip):
    q, k, v, skip = _projections(
        x,
        W_fc.T, b_fc.reshape(1, _D),
        Wq.T, bq.reshape(1, _D),
        Wk.T, bk.reshape(1, _D),
        Wv.T, bv.reshape(1, _D),
        Wskip.T, bskip.reshape(1, _D),
    )
    src = edge_index[0]
    dst = edge_index[1]
    nd = _edge_phase(q, k, v, src, dst)
    return _finalize(nd[0], nd[1], skip)


# speedup vs baseline: 8.4289x; 8.4289x over previous
import dataclasses
import math

import jax
import jax.numpy as jnp
from jax import lax
from jax.experimental import pallas as pl
from jax.experimental.pallas import tpu as pltpu
from jax.experimental.pallas import tpu_sc as plsc

_N = 10000
_E = 320000
_D = 128
_INV_SQRT_D = 1.0 / math.sqrt(128.0)

_NC = 2
_NS = 16
_NW = _NC * _NS
_EPW = _E // _NW
_C = 64
_NCHUNKS = _E // _C
_FULLROUNDS = _NCHUNKS // _NW
_EXTRA = _NCHUNKS % _NW
_AW = 128

_ROWBLK = 1000


def _proj_body(x_ref, wfc, bfc, wq, bq, wk, bk, wv, bv, ws, bs,
               q_ref, k_ref, v_ref, skip_ref):
    x = x_ref[...]
    h = jnp.maximum(
        jnp.dot(x, wfc[...], preferred_element_type=jnp.float32) + bfc[...],
        0.0)
    q_ref[...] = jnp.dot(h, wq[...], preferred_element_type=jnp.float32) + bq[...]
    k_ref[...] = jnp.dot(h, wk[...], preferred_element_type=jnp.float32) + bk[...]
    v_ref[...] = jnp.dot(h, wv[...], preferred_element_type=jnp.float32) + bv[...]
    skip_ref[...] = jnp.dot(h, ws[...], preferred_element_type=jnp.float32) + bs[...]


def _projections(x, wfc_t, bfc, wq_t, bq, wk_t, bk, wv_t, bv, ws_t, bs):
    w_spec = pl.BlockSpec((_D, _D), lambda i: (0, 0))
    b_spec = pl.BlockSpec((1, _D), lambda i: (0, 0))
    r_spec = pl.BlockSpec((_ROWBLK, _D), lambda i: (i, 0))
    out_sd = jax.ShapeDtypeStruct((_N, _D), jnp.float32)
    return pl.pallas_call(
        _proj_body,
        grid=(_N // _ROWBLK,),
        in_specs=[r_spec] + [w_spec, b_spec] * 5,
        out_specs=[r_spec] * 4,
        out_shape=[out_sd] * 4,
    )(x, wfc_t, bfc, wq_t, bq, wk_t, bk, wv_t, bv, ws_t, bs)


def _edge_body(q_hbm, k_hbm, v_hbm, ei_hbm, num_hbm, den_hbm,
               qr, kr, vr, sr, srcc, dstc, acc, accd):
    cid = lax.axis_index("c")
    sid = lax.axis_index("s")
    wid = cid * _NS + sid
    lane = lax.iota(jnp.int32, 16)

    @pl.loop(0, _C)
    def _(e):
        for c in range(8):
            sr[e, pl.ds(c * 16, 16)] = jnp.zeros((16,), jnp.float32)

    @pl.loop(0, 80)
    def _(r):
        for c in range(8):
            accd[r, pl.ds(c * 16, 16)] = jnp.zeros((16,), jnp.float32)

    @pl.loop(0, 39)
    def _(i):
        r0 = sid * 624 + i * 16
        pltpu.sync_copy(sr.at[pl.ds(0, 16)], acc.at[pl.ds(r0, 16)])

    @pl.when(sid == 0)
    def _():
        pltpu.sync_copy(sr.at[pl.ds(0, 16)], acc.at[pl.ds(9984, 16)])

    plsc.subcore_barrier()

    def do_chunk(off):
        pltpu.sync_copy(ei_hbm.at[pl.ds(off, _C)], srcc)
        pltpu.sync_copy(ei_hbm.at[pl.ds(_E + off, _C)], dstc)
        pltpu.sync_copy(q_hbm.at[dstc], qr)
        pltpu.sync_copy(k_hbm.at[srcc], kr)
        pltpu.sync_copy(v_hbm.at[srcc], vr)

        @pl.loop(0, _C // 16)
        def _(g):
            lvec = jnp.zeros((16,), jnp.float32)
            for j in range(16):
                e = g * 16 + j
                p = qr[e, pl.ds(0, 16)] * kr[e, pl.ds(0, 16)]
                for c in range(1, 8):
                    p = p + qr[e, pl.ds(c * 16, 16)] * kr[e, pl.ds(c * 16, 16)]
                s = jnp.sum(p)
                lvec = lvec + jnp.where(lane == j, s, 0.0)
            exv = jnp.exp(lvec * _INV_SQRT_D)
            dv = dstc[pl.ds(g * 16, 16)]
            drow = lax.shift_right_logical(dv, 7)
            dlane = lax.bitwise_and(dv, 127)
            for j in range(16):
                e = g * 16 + j
                ex = exv[j]
                for c in range(8):
                    sr[e, pl.ds(c * 16, 16)] = vr[e, pl.ds(c * 16, 16)] * ex
                plsc.addupdate_scatter(accd, [drow, dlane], exv,
                                       mask=lane == j)

        pltpu.sync_copy(sr, acc.at[dstc], add=True)

    @pl.loop(0, _FULLROUNDS)
    def _(i):
        do_chunk((wid + _NW * i) * _C)

    if _EXTRA:
        @pl.when(wid < _EXTRA)
        def _():
            do_chunk((wid + _NW * _FULLROUNDS) * _C)

    plsc.subcore_barrier()

    @pl.loop(0, 39)
    def _(i):
        r0 = sid * 624 + i * 16
        pltpu.sync_copy(acc.at[pl.ds(r0, 16)],
                        num_hbm.at[cid].at[pl.ds(r0, 16)])

    pltpu.sync_copy(accd, den_hbm.at[wid])

    @pl.when(sid == 0)
    def _():
        pltpu.sync_copy(acc.at[pl.ds(9984, 16)],
                        num_hbm.at[cid].at[pl.ds(9984, 16)])


def _edge_phase(q, k, v, edge_index):
    mesh = plsc.VectorSubcoreMesh(core_axis_name="c", subcore_axis_name="s")
    cp = pltpu.CompilerParams()
    if "needs_layout_passes" in pltpu.CompilerParams.__dataclass_fields__:
        cp = dataclasses.replace(cp, needs_layout_passes=False)
    ker = pl.kernel(
        _edge_body,
        out_type=[
            jax.ShapeDtypeStruct((_NC, _N, _D), jnp.float32),
            jax.ShapeDtypeStruct((_NW, 80, _D), jnp.float32),
        ],
        mesh=mesh,
        scratch_types=[
            pltpu.VMEM((_C, _D), jnp.float32),
            pltpu.VMEM((_C, _D), jnp.float32),
            pltpu.VMEM((_C, _D), jnp.float32),
            pltpu.VMEM((_C, _D), jnp.float32),
            pltpu.VMEM((_C,), jnp.int32),
            pltpu.VMEM((_C,), jnp.int32),
            pltpu.VMEM_SHARED((_N, _D), jnp.float32),
            pltpu.VMEM((80, _D), jnp.float32),
        ],
        compiler_params=cp,
    )
    return ker(q, k, v, edge_index)


def _final_body(n0_ref, n1_ref, d_ref, skip_ref, o_ref):
    num = n0_ref[...] + n1_ref[...]
    den = jnp.sum(d_ref[...], axis=1, keepdims=True)
    o_ref[...] = num / (den + 1e-16) + skip_ref[...]


def _finalize(n0, n1, den_t, skip):
    d_spec = pl.BlockSpec((_ROWBLK, _NW), lambda i: (i, 0))
    r_spec = pl.BlockSpec((_ROWBLK, _D), lambda i: (i, 0))
    return pl.pallas_call(
        _final_body,
        grid=(_N // _ROWBLK,),
        in_specs=[r_spec, r_spec, d_spec, r_spec],
        out_specs=r_spec,
        out_shape=jax.ShapeDtypeStruct((_N, _D), jnp.float32),
    )(n0, n1, den_t, skip)


@jax.jit
def kernel(x, edge_index, W_fc, b_fc, Wq, bq, Wk, bk, Wv, bv, Wskip, bskip):
    q, k, v, skip = _projections(
        x,
        W_fc.T, b_fc.reshape(1, _D),
        Wq.T, bq.reshape(1, _D),
        Wk.T, bk.reshape(1, _D),
        Wv.T, bv.reshape(1, _D),
        Wskip.T, bskip.reshape(1, _D),
    )
    num, den = _edge_phase(q, k, v, edge_index.reshape(2 * _E))
    den_t = den.reshape(_NW, 80 * _D)[:, :_N].T
    return _finalize(num[0], num[1], den_t, skip)


# speedup vs baseline: 12.2806x; 1.4570x over previous
import dataclasses
import math

import jax
import jax.numpy as jnp
from jax import lax
from jax.experimental import pallas as pl
from jax.experimental.pallas import tpu as pltpu
from jax.experimental.pallas import tpu_sc as plsc

_N = 10000
_E = 320000
_D = 128
_INV_SQRT_D = 1.0 / math.sqrt(128.0)

_NC = 2
_NS = 16
_NW = _NC * _NS
_EPW = _E // _NW
_C = 64
_NCHUNKS = _E // _C
_FULLROUNDS = _NCHUNKS // _NW
_EXTRA = _NCHUNKS % _NW
_AW = 128

_ROWBLK = 1000


def _proj_body(x_ref, wfc, bfc, wq, bq, wk, bk, wv, bv, ws, bs,
               q_ref, k_ref, v_ref, skip_ref):
    x = x_ref[...]
    h = jnp.maximum(
        jnp.dot(x, wfc[...], preferred_element_type=jnp.float32) + bfc[...],
        0.0)
    q_ref[...] = jnp.dot(h, wq[...], preferred_element_type=jnp.float32) + bq[...]
    k_ref[...] = jnp.dot(h, wk[...], preferred_element_type=jnp.float32) + bk[...]
    v_ref[...] = jnp.dot(h, wv[...], preferred_element_type=jnp.float32) + bv[...]
    skip_ref[...] = jnp.dot(h, ws[...], preferred_element_type=jnp.float32) + bs[...]


def _projections(x, wfc_t, bfc, wq_t, bq, wk_t, bk, wv_t, bv, ws_t, bs):
    w_spec = pl.BlockSpec((_D, _D), lambda i: (0, 0))
    b_spec = pl.BlockSpec((1, _D), lambda i: (0, 0))
    r_spec = pl.BlockSpec((_ROWBLK, _D), lambda i: (i, 0))
    out_sd = jax.ShapeDtypeStruct((_N, _D), jnp.float32)
    return pl.pallas_call(
        _proj_body,
        grid=(_N // _ROWBLK,),
        in_specs=[r_spec] + [w_spec, b_spec] * 5,
        out_specs=[r_spec] * 4,
        out_shape=[out_sd] * 4,
    )(x, wfc_t, bfc, wq_t, bq, wk_t, bk, wv_t, bv, ws_t, bs)


def _edge_body(q_hbm, k_hbm, v_hbm, ei_hbm, num_hbm, den_hbm,
               qr, kr, vr, sr, srcc, dstc, dsc, acc, accd,
               sem_i, sem_g, sem_s):
    cid = lax.axis_index("c")
    sid = lax.axis_index("s")
    wid = cid * _NS + sid
    lane = lax.iota(jnp.int32, 16)

    @pl.loop(0, _C)
    def _(e):
        for c in range(8):
            sr[e, pl.ds(c * 16, 16)] = jnp.zeros((16,), jnp.float32)

    @pl.loop(0, 80)
    def _(r):
        for c in range(8):
            accd[r, pl.ds(c * 16, 16)] = jnp.zeros((16,), jnp.float32)

    @pl.loop(0, 39)
    def _(i):
        r0 = sid * 624 + i * 16
        pltpu.sync_copy(sr.at[pl.ds(0, 16)], acc.at[pl.ds(r0, 16)])

    @pl.when(sid == 0)
    def _():
        pltpu.sync_copy(sr.at[pl.ds(0, 16)], acc.at[pl.ds(9984, 16)])

    plsc.subcore_barrier()

    def drain_scatter():
        pltpu.make_async_copy(num_hbm.at[cid].at[pl.ds(0, _C)], sr,
                              sem_s).wait()

    def do_chunk(off, not_first):
        hi0 = pltpu.async_copy(ei_hbm.at[pl.ds(off, _C)], srcc, sem_i)
        hi1 = pltpu.async_copy(ei_hbm.at[pl.ds(_E + off, _C)], dstc, sem_i)
        hi0.wait()
        hi1.wait()
        hg0 = pltpu.async_copy(q_hbm.at[dstc], qr, sem_g)
        hg1 = pltpu.async_copy(k_hbm.at[srcc], kr, sem_g)
        hg2 = pltpu.async_copy(v_hbm.at[srcc], vr, sem_g)

        @pl.when(not_first)
        def _():
            drain_scatter()

        hg0.wait()
        hg1.wait()
        hg2.wait()

        @pl.loop(0, _C // 16)
        def _(g):
            dsc[pl.ds(g * 16, 16)] = dstc[pl.ds(g * 16, 16)]

        @pl.loop(0, _C // 16)
        def _(g):
            lvec = jnp.zeros((16,), jnp.float32)
            for j in range(16):
                e = g * 16 + j
                p = qr[e, pl.ds(0, 16)] * kr[e, pl.ds(0, 16)]
                for c in range(1, 8):
                    p = p + qr[e, pl.ds(c * 16, 16)] * kr[e, pl.ds(c * 16, 16)]
                s = jnp.sum(p)
                lvec = lvec + jnp.where(lane == j, s, 0.0)
            exv = jnp.exp(lvec * _INV_SQRT_D)
            dv = dstc[pl.ds(g * 16, 16)]
            drow = lax.shift_right_logical(dv, 7)
            dlane = lax.bitwise_and(dv, 127)
            for j in range(16):
                e = g * 16 + j
                ex = exv[j]
                for c in range(8):
                    sr[e, pl.ds(c * 16, 16)] = vr[e, pl.ds(c * 16, 16)] * ex
                plsc.addupdate_scatter(accd, [drow, dlane], exv,
                                       mask=lane == j)

        pltpu.async_copy(sr, acc.at[dsc], add=True, sem=sem_s)

    @pl.loop(0, _FULLROUNDS)
    def _(i):
        do_chunk((wid + _NW * i) * _C, i > 0)

    if _EXTRA:
        @pl.when(wid < _EXTRA)
        def _():
            do_chunk((wid + _NW * _FULLROUNDS) * _C, True)

    drain_scatter()

    plsc.subcore_barrier()

    @pl.loop(0, 39)
    def _(i):
        r0 = sid * 624 + i * 16
        pltpu.sync_copy(acc.at[pl.ds(r0, 16)],
                        num_hbm.at[cid].at[pl.ds(r0, 16)])

    pltpu.sync_copy(accd, den_hbm.at[wid])

    @pl.when(sid == 0)
    def _():
        pltpu.sync_copy(acc.at[pl.ds(9984, 16)],
                        num_hbm.at[cid].at[pl.ds(9984, 16)])


def _edge_phase(q, k, v, edge_index):
    mesh = plsc.VectorSubcoreMesh(core_axis_name="c", subcore_axis_name="s")
    cp = pltpu.CompilerParams()
    if "needs_layout_passes" in pltpu.CompilerParams.__dataclass_fields__:
        cp = dataclasses.replace(cp, needs_layout_passes=False)
    ker = pl.kernel(
        _edge_body,
        out_type=[
            jax.ShapeDtypeStruct((_NC, _N, _D), jnp.float32),
            jax.ShapeDtypeStruct((_NW, 80, _D), jnp.float32),
        ],
        mesh=mesh,
        scratch_types=[
            pltpu.VMEM((_C, _D), jnp.float32),
            pltpu.VMEM((_C, _D), jnp.float32),
            pltpu.VMEM((_C, _D), jnp.float32),
            pltpu.VMEM((_C, _D), jnp.float32),
            pltpu.VMEM((_C,), jnp.int32),
            pltpu.VMEM((_C,), jnp.int32),
            pltpu.VMEM((_C,), jnp.int32),
            pltpu.VMEM_SHARED((_N, _D), jnp.float32),
            pltpu.VMEM((80, _D), jnp.float32),
            pltpu.SemaphoreType.DMA,
            pltpu.SemaphoreType.DMA,
            pltpu.SemaphoreType.DMA,
        ],
        compiler_params=cp,
    )
    return ker(q, k, v, edge_index)


def _final_body(n0_ref, n1_ref, d_ref, skip_ref, o_ref):
    num = n0_ref[...] + n1_ref[...]
    den = jnp.sum(d_ref[...], axis=1, keepdims=True)
    o_ref[...] = num / (den + 1e-16) + skip_ref[...]


def _finalize(n0, n1, den_t, skip):
    d_spec = pl.BlockSpec((_ROWBLK, _NW), lambda i: (i, 0))
    r_spec = pl.BlockSpec((_ROWBLK, _D), lambda i: (i, 0))
    return pl.pallas_call(
        _final_body,
        grid=(_N // _ROWBLK,),
        in_specs=[r_spec, r_spec, d_spec, r_spec],
        out_specs=r_spec,
        out_shape=jax.ShapeDtypeStruct((_N, _D), jnp.float32),
    )(n0, n1, den_t, skip)


@jax.jit
def kernel(x, edge_index, W_fc, b_fc, Wq, bq, Wk, bk, Wv, bv, Wskip, bskip):
    q, k, v, skip = _projections(
        x,
        W_fc.T, b_fc.reshape(1, _D),
        Wq.T, bq.reshape(1, _D),
        Wk.T, bk.reshape(1, _D),
        Wv.T, bv.reshape(1, _D),
        Wskip.T, bskip.reshape(1, _D),
    )
    num, den = _edge_phase(q, k, v, edge_index.reshape(2 * _E))
    den_t = den.reshape(_NW, 80 * _D)[:, :_N].T
    return _finalize(num[0], num[1], den_t, skip)


# speedup vs baseline: 16.8535x; 1.3724x over previous
import dataclasses
import math

import jax
import jax.numpy as jnp
from jax import lax
from jax.experimental import pallas as pl
from jax.experimental.pallas import tpu as pltpu
from jax.experimental.pallas import tpu_sc as plsc

_N = 10000
_E = 320000
_D = 128
_INV_SQRT_D = 1.0 / math.sqrt(128.0)

_NC = 2
_NS = 16
_NW = _NC * _NS
_EPW = _E // _NW
_C = 32
_NCHUNKS = _E // _C
_FULLROUNDS = _NCHUNKS // _NW
_EXTRA = _NCHUNKS % _NW
_AW = 128

_ROWBLK = 1000


def _proj_body(x_ref, wfc, bfc, wq, bq, wk, bk, wv, bv, ws, bs,
               q_ref, k_ref, v_ref, skip_ref):
    x = x_ref[...]
    h = jnp.maximum(
        jnp.dot(x, wfc[...], preferred_element_type=jnp.float32) + bfc[...],
        0.0)
    q_ref[...] = jnp.dot(h, wq[...], preferred_element_type=jnp.float32) + bq[...]
    k_ref[...] = jnp.dot(h, wk[...], preferred_element_type=jnp.float32) + bk[...]
    v_ref[...] = jnp.dot(h, wv[...], preferred_element_type=jnp.float32) + bv[...]
    skip_ref[...] = jnp.dot(h, ws[...], preferred_element_type=jnp.float32) + bs[...]


def _projections(x, wfc_t, bfc, wq_t, bq, wk_t, bk, wv_t, bv, ws_t, bs):
    w_spec = pl.BlockSpec((_D, _D), lambda i: (0, 0))
    b_spec = pl.BlockSpec((1, _D), lambda i: (0, 0))
    r_spec = pl.BlockSpec((_ROWBLK, _D), lambda i: (i, 0))
    out_sd = jax.ShapeDtypeStruct((_N, _D), jnp.float32)
    return pl.pallas_call(
        _proj_body,
        grid=(_N // _ROWBLK,),
        in_specs=[r_spec] + [w_spec, b_spec] * 5,
        out_specs=[r_spec] * 4,
        out_shape=[out_sd] * 4,
    )(x, wfc_t, bfc, wq_t, bq, wk_t, bk, wv_t, bv, ws_t, bs)


def _edge_body(q_hbm, k_hbm, v_hbm, ei_hbm, num_hbm, den_hbm,
               qr0, kr0, vr0, qr1, kr1, vr1,
               src0, dst0, src1, dst1, dsc0, dsc1, acc, accd,
               sem_i0, sem_i1, sem_g0, sem_g1, sem_s):
    cid = lax.axis_index("c")
    sid = lax.axis_index("s")
    wid = cid * _NS + sid
    lane = lax.iota(jnp.int32, 16)
    nrounds = _FULLROUNDS + jnp.where(wid < _EXTRA, 1, 0)

    @pl.loop(0, 80)
    def _(r):
        for c in range(8):
            accd[r, pl.ds(c * 16, 16)] = jnp.zeros((16,), jnp.float32)

    @pl.loop(0, 39)
    def _(i):
        r0 = sid * 624 + i * 16
        pltpu.sync_copy(accd.at[pl.ds(0, 16)], acc.at[pl.ds(r0, 16)])

    @pl.when(sid == 0)
    def _():
        pltpu.sync_copy(accd.at[pl.ds(0, 16)], acc.at[pl.ds(9984, 16)])

    plsc.subcore_barrier()

    def off_of(j):
        return (wid + _NW * j) * _C

    def issue_idx(j, srcb, dstb, sem):
        pltpu.async_copy(ei_hbm.at[pl.ds(off_of(j), _C)], srcb, sem)
        pltpu.async_copy(ei_hbm.at[pl.ds(_E + off_of(j), _C)], dstb, sem)

    def wait_idx(j, srcb, dstb, sem):
        pltpu.make_async_copy(ei_hbm.at[pl.ds(off_of(j), _C)], srcb, sem).wait()
        pltpu.make_async_copy(ei_hbm.at[pl.ds(_E + off_of(j), _C)], dstb,
                              sem).wait()

    def issue_gathers(srcb, dstb, qr, kr, vr, sem):
        pltpu.async_copy(q_hbm.at[dstb], qr, sem)
        pltpu.async_copy(k_hbm.at[srcb], kr, sem)
        pltpu.async_copy(v_hbm.at[srcb], vr, sem)

    def wait_gathers(srcb, dstb, qr, kr, vr, sem):
        pltpu.make_async_copy(q_hbm.at[dstb], qr, sem).wait()
        pltpu.make_async_copy(k_hbm.at[srcb], kr, sem).wait()
        pltpu.make_async_copy(v_hbm.at[srcb], vr, sem).wait()

    def drain_scatter():
        pltpu.make_async_copy(num_hbm.at[cid].at[pl.ds(0, _C)], vr0,
                              sem_s).wait()

    def compute_and_scatter(qr, kr, vr, dstb, dsc):
        @pl.loop(0, _C // 16)
        def _(g):
            dsc[pl.ds(g * 16, 16)] = dstb[pl.ds(g * 16, 16)]

        @pl.loop(0, _C // 16)
        def _(g):
            lvec = jnp.zeros((16,), jnp.float32)
            for j in range(16):
                e = g * 16 + j
                p = qr[e, pl.ds(0, 16)] * kr[e, pl.ds(0, 16)]
                for c in range(1, 8):
                    p = p + qr[e, pl.ds(c * 16, 16)] * kr[e, pl.ds(c * 16, 16)]
                s = jnp.sum(p)
                lvec = lvec + jnp.where(lane == j, s, 0.0)
            exv = jnp.exp(lvec * _INV_SQRT_D)
            dv = dsc[pl.ds(g * 16, 16)]
            drow = lax.shift_right_logical(dv, 7)
            dlane = lax.bitwise_and(dv, 127)
            for j in range(16):
                e = g * 16 + j
                ex = exv[j]
                for c in range(8):
                    vr[e, pl.ds(c * 16, 16)] = vr[e, pl.ds(c * 16, 16)] * ex
                plsc.addupdate_scatter(accd, [drow, dlane], exv,
                                       mask=lane == j)

        pltpu.async_copy(vr, acc.at[dsc], add=True, sem=sem_s)

    bufs = (
        (qr0, kr0, vr0, src0, dst0, dsc0, sem_i0, sem_g0),
        (qr1, kr1, vr1, src1, dst1, dsc1, sem_i1, sem_g1),
    )

    issue_idx(0, src0, dst0, sem_i0)
    wait_idx(0, src0, dst0, sem_i0)
    issue_gathers(src0, dst0, qr0, kr0, vr0, sem_g0)
    issue_idx(1, src1, dst1, sem_i1)

    @pl.loop(0, _FULLROUNDS + (1 if _EXTRA else 0))
    def _(i):
        def round_body(cur, nxt):
            qr, kr, vr, srcb, dstb, dsc, sem_i, sem_g = cur
            qrn, krn, vrn, srcn, dstn, dscn, sem_in, sem_gn = nxt

            @pl.when(i > 0)
            def _():
                drain_scatter()

            @pl.when(i < nrounds)
            def _():
                wait_gathers(srcb, dstb, qr, kr, vr, sem_g)

            @pl.when(i + 1 < nrounds)
            def _():
                wait_idx(i + 1, srcn, dstn, sem_in)
                issue_gathers(srcn, dstn, qrn, krn, vrn, sem_gn)

            @pl.when(i < nrounds)
            def _():
                compute_and_scatter(qr, kr, vr, dstb, dsc)

            @pl.when(i + 2 < nrounds)
            def _():
                issue_idx(i + 2, srcb, dstb, sem_i)

        par = lax.rem(i, 2)

        @pl.when(par == 0)
        def _():
            round_body(bufs[0], bufs[1])

        @pl.when(par == 1)
        def _():
            round_body(bufs[1], bufs[0])

    if _EXTRA:
        @pl.when(wid < _EXTRA)
        def _():
            drain_scatter()
    else:
        drain_scatter()

    plsc.subcore_barrier()

    @pl.loop(0, 39)
    def _(i):
        r0 = sid * 624 + i * 16
        pltpu.sync_copy(acc.at[pl.ds(r0, 16)],
                        num_hbm.at[cid].at[pl.ds(r0, 16)])

    pltpu.sync_copy(accd, den_hbm.at[wid])

    @pl.when(sid == 0)
    def _():
        pltpu.sync_copy(acc.at[pl.ds(9984, 16)],
                        num_hbm.at[cid].at[pl.ds(9984, 16)])


def _edge_phase(q, k, v, edge_index):
    mesh = plsc.VectorSubcoreMesh(core_axis_name="c", subcore_axis_name="s")
    cp = pltpu.CompilerParams()
    if "needs_layout_passes" in pltpu.CompilerParams.__dataclass_fields__:
        cp = dataclasses.replace(cp, needs_layout_passes=False)
    ker = pl.kernel(
        _edge_body,
        out_type=[
            jax.ShapeDtypeStruct((_NC, _N, _D), jnp.float32),
            jax.ShapeDtypeStruct((_NW, 80, _D), jnp.float32),
        ],
        mesh=mesh,
        scratch_types=[
            pltpu.VMEM((_C, _D), jnp.float32),
            pltpu.VMEM((_C, _D), jnp.float32),
            pltpu.VMEM((_C, _D), jnp.float32),
            pltpu.VMEM((_C, _D), jnp.float32),
            pltpu.VMEM((_C, _D), jnp.float32),
            pltpu.VMEM((_C, _D), jnp.float32),
            pltpu.VMEM((_C,), jnp.int32),
            pltpu.VMEM((_C,), jnp.int32),
            pltpu.VMEM((_C,), jnp.int32),
            pltpu.VMEM((_C,), jnp.int32),
            pltpu.VMEM((_C,), jnp.int32),
            pltpu.VMEM((_C,), jnp.int32),
            pltpu.VMEM_SHARED((_N, _D), jnp.float32),
            pltpu.VMEM((80, _D), jnp.float32),
            pltpu.SemaphoreType.DMA,
            pltpu.SemaphoreType.DMA,
            pltpu.SemaphoreType.DMA,
            pltpu.SemaphoreType.DMA,
            pltpu.SemaphoreType.DMA,
        ],
        compiler_params=cp,
    )
    return ker(q, k, v, edge_index)


def _final_body(n0_ref, n1_ref, d_ref, skip_ref, o_ref):
    num = n0_ref[...] + n1_ref[...]
    den = jnp.sum(d_ref[...], axis=1, keepdims=True)
    o_ref[...] = num / (den + 1e-16) + skip_ref[...]


def _finalize(n0, n1, den_t, skip):
    d_spec = pl.BlockSpec((_ROWBLK, _NW), lambda i: (i, 0))
    r_spec = pl.BlockSpec((_ROWBLK, _D), lambda i: (i, 0))
    return pl.pallas_call(
        _final_body,
        grid=(_N // _ROWBLK,),
        in_specs=[r_spec, r_spec, d_spec, r_spec],
        out_specs=r_spec,
        out_shape=jax.ShapeDtypeStruct((_N, _D), jnp.float32),
    )(n0, n1, den_t, skip)


@jax.jit
def kernel(x, edge_index, W_fc, b_fc, Wq, bq, Wk, bk, Wv, bv, Wskip, bskip):
    q, k, v, skip = _projections(
        x,
        W_fc.T, b_fc.reshape(1, _D),
        Wq.T, bq.reshape(1, _D),
        Wk.T, bk.reshape(1, _D),
        Wv.T, bv.reshape(1, _D),
        Wskip.T, bskip.reshape(1, _D),
    )
    num, den = _edge_phase(q, k, v, edge_index.reshape(2 * _E))
    den_t = den.reshape(_NW, 80 * _D)[:, :_N].T
    return _finalize(num[0], num[1], den_t, skip)
